# Initial kernel scaffold; baseline (speedup 1.0000x reference)
import jax
import jax.numpy as jnp
from jax.experimental import pallas as pl


def kernel(x, edge_index, W1, b1, W2, b2):
    raise NotImplementedError("write your pallas kernel here")


# speedup vs baseline: 7.7007x; 7.7007x over previous
import functools

import jax
import jax.numpy as jnp
from jax import lax
from jax.experimental import pallas as pl
from jax.experimental.pallas import tpu as pltpu
from jax.experimental.pallas import tpu_sc as plsc

N = 10000
E = 320000
HID = 256
OUT = 128
K_ITERS = 10
ALPHA = 0.1

NC = 2
NT = 16
HC = OUT // NC
EPT = E // NT
CHUNK = 128
NCHUNK = 160
PAD_E = NCHUNK * CHUNK - EPT
NP = 10240
RPT = NP // NT
SUB = RPT // 8
NBUF = 2


def _prop_body(s_in, p9b, s0, rows_h, cols_h, s_out,
               agg_sp, rows_v, cols_v, gb0, gb1, ub_agg, ub_p, ub_s0,
               gsem, ssem):
    c = lax.axis_index("c")
    t = lax.axis_index("s")
    r0 = t * RPT
    plane = c * NP
    gbufs = (gb0, gb1)

    pltpu.sync_copy(rows_h.at[t], rows_v)
    pltpu.sync_copy(cols_h.at[t], cols_v)

    def adj_body(j, _):
        for k in range(CHUNK // 16):
            sl = pl.ds(k * 16, 16)
            rows_v[j, sl] = rows_v[j, sl] + plane
        return 0
    lax.fori_loop(0, NCHUNK, adj_body, 0)

    pltpu.sync_copy(s_in.at[pl.ds(plane + r0, RPT)], agg_sp.at[pl.ds(r0, RPT)])
    plsc.subcore_barrier()

    def edge_group(jj, _):
        for b in range(NBUF):
            j = jj * NBUF + b

            @pl.when(jj > 0)
            def _wait_prev():
                pltpu.make_async_copy(
                    gbufs[b], agg_sp.at[cols_v.at[j]], ssem.at[b]).wait()

            pltpu.async_copy(s_in.at[rows_v.at[j]], gbufs[b], gsem.at[b])
        for b in range(NBUF):
            j = jj * NBUF + b
            pltpu.make_async_copy(
                s_in.at[rows_v.at[j]], gbufs[b], gsem.at[b]).wait()
            pltpu.async_copy(
                gbufs[b], agg_sp.at[cols_v.at[j]], ssem.at[b], add=True)
        return 0
    lax.fori_loop(0, NCHUNK // NBUF, edge_group, 0)
    for b in range(NBUF):
        j = NCHUNK - NBUF + b
        pltpu.make_async_copy(
            gbufs[b], agg_sp.at[cols_v.at[j]], ssem.at[b]).wait()
    plsc.subcore_barrier()

    for u in range(RPT // SUB):
        ur = r0 + u * SUB
        pltpu.sync_copy(agg_sp.at[pl.ds(ur, SUB)], ub_agg)
        pltpu.sync_copy(p9b.at[pl.ds(ur, SUB)], ub_p)
        pltpu.sync_copy(s0.at[pl.ds(plane + ur, SUB)], ub_s0)

        def upd_body(r, _):
            for k in range(HC // 16):
                sl = pl.ds(k * 16, 16)
                ub_agg[r, sl] = (ub_p[r, sl] * ub_agg[r, sl]
                                 + 0.1 * ub_s0[r, sl])
            return 0
        lax.fori_loop(0, SUB, upd_body, 0)
        pltpu.sync_copy(ub_agg, s_out.at[pl.ds(plane + ur, SUB)])


_prop = functools.partial(
    pl.kernel,
    out_type=jax.ShapeDtypeStruct((NC * NP, HC), jnp.float32),
    mesh=plsc.VectorSubcoreMesh(core_axis_name="c", subcore_axis_name="s"),
    scratch_types=[
        pltpu.VMEM_SHARED((NP, HC), jnp.float32),
        pltpu.VMEM((NCHUNK, CHUNK), jnp.int32),
        pltpu.VMEM((NCHUNK, CHUNK), jnp.int32),
        pltpu.VMEM((CHUNK, HC), jnp.float32),
        pltpu.VMEM((CHUNK, HC), jnp.float32),
        pltpu.VMEM((SUB, HC), jnp.float32),
        pltpu.VMEM((SUB, HC), jnp.float32),
        pltpu.VMEM((SUB, HC), jnp.float32),
        pltpu.SemaphoreType.DMA((NBUF,)),
        pltpu.SemaphoreType.DMA((NBUF,)),
    ],
    compiler_params=pltpu.CompilerParams(use_tc_tiling_on_sc=False),
)(_prop_body)


BN = 1000


def _prep_body(x_ref, w1_ref, b1_ref, w2_ref, b2_ref, deg_ref,
               sa_ref, sb_ref, p9_ref, sq_ref):
    h1 = lax.dot_general(x_ref[...], w1_ref[...], (((1,), (1,)), ((), ())),
                         preferred_element_type=jnp.float32)
    h1 = jnp.maximum(h1 + b1_ref[...], 0.0)
    h = lax.dot_general(h1, w2_ref[...], (((1,), (1,)), ((), ())),
                        preferred_element_type=jnp.float32)
    h = h + b2_ref[...]
    deg = deg_ref[...]
    dinv = lax.rsqrt(deg)
    s0 = h * dinv
    sa_ref[...] = s0[:, :HC]
    sb_ref[...] = s0[:, HC:]
    p9_ref[...] = jnp.broadcast_to((1.0 - ALPHA) / deg, (BN, HC))
    sq_ref[...] = deg * dinv


def _prep(x, w1, b1, w2, b2, deg):
    return pl.pallas_call(
        _prep_body,
        grid=(N // BN,),
        in_specs=[
            pl.BlockSpec((BN, OUT), lambda i: (i, 0)),
            pl.BlockSpec((HID, OUT), lambda i: (0, 0)),
            pl.BlockSpec((1, HID), lambda i: (0, 0)),
            pl.BlockSpec((OUT, HID), lambda i: (0, 0)),
            pl.BlockSpec((1, OUT), lambda i: (0, 0)),
            pl.BlockSpec((BN, 1), lambda i: (i, 0)),
        ],
        out_specs=[
            pl.BlockSpec((BN, HC), lambda i: (i, 0)),
            pl.BlockSpec((BN, HC), lambda i: (i, 0)),
            pl.BlockSpec((BN, HC), lambda i: (i, 0)),
            pl.BlockSpec((BN, 1), lambda i: (i, 0)),
        ],
        out_shape=[
            jax.ShapeDtypeStruct((N, HC), jnp.float32),
            jax.ShapeDtypeStruct((N, HC), jnp.float32),
            jax.ShapeDtypeStruct((N, HC), jnp.float32),
            jax.ShapeDtypeStruct((N, 1), jnp.float32),
        ],
    )(x, w1, b1, w2, b2, deg)


def _final_body(fa_ref, fb_ref, sq_ref, out_ref):
    y = jnp.concatenate([fa_ref[...], fb_ref[...]], axis=1) * sq_ref[...]
    m = jnp.max(y, axis=1, keepdims=True)
    e = jnp.exp(y - m)
    lse = jnp.log(jnp.sum(e, axis=1, keepdims=True)) + m
    out_ref[...] = y - lse


def _final(fa, fb, sq):
    return pl.pallas_call(
        _final_body,
        grid=(N // BN,),
        in_specs=[
            pl.BlockSpec((BN, HC), lambda i: (i, 0)),
            pl.BlockSpec((BN, HC), lambda i: (i, 0)),
            pl.BlockSpec((BN, 1), lambda i: (i, 0)),
        ],
        out_specs=pl.BlockSpec((BN, OUT), lambda i: (i, 0)),
        out_shape=jax.ShapeDtypeStruct((N, OUT), jnp.float32),
    )(fa, fb, sq)


def kernel(x, edge_index, W1, b1, W2, b2):
    f32 = jnp.float32
    rows = edge_index[0].astype(jnp.int32).reshape(NT, EPT)
    cols = edge_index[1].astype(jnp.int32).reshape(NT, EPT)
    rows = jnp.pad(rows, ((0, 0), (0, PAD_E))).reshape(NT, NCHUNK, CHUNK)
    cols = jnp.pad(cols, ((0, 0), (0, PAD_E)),
                   constant_values=N).reshape(NT, NCHUNK, CHUNK)

    deg_full = _prop(jnp.ones((NC * NP, HC), f32), jnp.ones((NP, HC), f32),
                     jnp.zeros((NC * NP, HC), f32), rows, cols)
    deg = deg_full[:N, 0:1]

    sa, sb, p9, sq = _prep(x, W1, b1.reshape(1, HID), W2, b2.reshape(1, OUT),
                           deg)

    pad_rows = ((0, NP - N), (0, 0))
    s0 = jnp.concatenate(
        [jnp.pad(sa, pad_rows), jnp.pad(sb, pad_rows)], axis=0)
    p9b = jnp.pad(p9, pad_rows)

    s = s0
    for _ in range(K_ITERS):
        s = _prop(s, p9b, s0, rows, cols)

    return _final(s[:N], s[NP:NP + N], sq)


# speedup vs baseline: 8.6541x; 1.1238x over previous
import functools

import jax
import jax.numpy as jnp
from jax import lax
from jax.experimental import pallas as pl
from jax.experimental.pallas import tpu as pltpu
from jax.experimental.pallas import tpu_sc as plsc

N = 10000
E = 320000
HID = 256
OUT = 128
K_ITERS = 10
ALPHA = 0.1

NC = 2
NT = 16
HC = OUT // NC
EPT = E // NT
CHUNK = 128
NCHUNK = 160
PAD_E = NCHUNK * CHUNK - EPT
NP = 10240
RPT = NP // NT
SUB = RPT // 16
NBUF = 4


def _prop_body(s_in, p9b, s0, rows_h, cols_h, s_out,
               agg_sp, rows_v, cols_v, gb0, gb1, gb2, gb3,
               ub_agg, ub_p, ub_s0, gsem, ssem):
    c = lax.axis_index("c")
    t = lax.axis_index("s")
    r0 = t * RPT
    plane = c * NP
    gbufs = (gb0, gb1, gb2, gb3)

    pltpu.sync_copy(rows_h.at[t], rows_v)
    pltpu.sync_copy(cols_h.at[t], cols_v)

    def adj_body(j, _):
        for k in range(CHUNK // 16):
            sl = pl.ds(k * 16, 16)
            rows_v[j, sl] = rows_v[j, sl] + plane
        return 0
    lax.fori_loop(0, NCHUNK, adj_body, 0)

    pltpu.sync_copy(s_in.at[pl.ds(plane + r0, RPT)], agg_sp.at[pl.ds(r0, RPT)])
    plsc.subcore_barrier()

    def edge_group(jj, _):
        for b in range(NBUF):
            j = jj * NBUF + b

            @pl.when(jj > 0)
            def _wait_prev():
                pltpu.make_async_copy(
                    gbufs[b], agg_sp.at[cols_v.at[j]], ssem.at[b]).wait()

            pltpu.async_copy(s_in.at[rows_v.at[j]], gbufs[b], gsem.at[b])
        for b in range(NBUF):
            j = jj * NBUF + b
            pltpu.make_async_copy(
                s_in.at[rows_v.at[j]], gbufs[b], gsem.at[b]).wait()
            pltpu.async_copy(
                gbufs[b], agg_sp.at[cols_v.at[j]], ssem.at[b], add=True)
        return 0
    lax.fori_loop(0, NCHUNK // NBUF, edge_group, 0)
    for b in range(NBUF):
        j = NCHUNK - NBUF + b
        pltpu.make_async_copy(
            gbufs[b], agg_sp.at[cols_v.at[j]], ssem.at[b]).wait()
    plsc.subcore_barrier()

    for u in range(RPT // SUB):
        ur = r0 + u * SUB
        pltpu.sync_copy(agg_sp.at[pl.ds(ur, SUB)], ub_agg)
        pltpu.sync_copy(p9b.at[pl.ds(ur, SUB)], ub_p)
        pltpu.sync_copy(s0.at[pl.ds(plane + ur, SUB)], ub_s0)

        def upd_body(r, _):
            for k in range(HC // 16):
                sl = pl.ds(k * 16, 16)
                ub_agg[r, sl] = (ub_p[r, sl] * ub_agg[r, sl]
                                 + 0.1 * ub_s0[r, sl])
            return 0
        lax.fori_loop(0, SUB, upd_body, 0)
        pltpu.sync_copy(ub_agg, s_out.at[pl.ds(plane + ur, SUB)])


_prop = functools.partial(
    pl.kernel,
    out_type=jax.ShapeDtypeStruct((NC * NP, HC), jnp.float32),
    mesh=plsc.VectorSubcoreMesh(core_axis_name="c", subcore_axis_name="s"),
    scratch_types=[
        pltpu.VMEM_SHARED((NP, HC), jnp.float32),
        pltpu.VMEM((NCHUNK, CHUNK), jnp.int32),
        pltpu.VMEM((NCHUNK, CHUNK), jnp.int32),
        pltpu.VMEM((CHUNK, HC), jnp.float32),
        pltpu.VMEM((CHUNK, HC), jnp.float32),
        pltpu.VMEM((CHUNK, HC), jnp.float32),
        pltpu.VMEM((CHUNK, HC), jnp.float32),
        pltpu.VMEM((SUB, HC), jnp.float32),
        pltpu.VMEM((SUB, HC), jnp.float32),
        pltpu.VMEM((SUB, HC), jnp.float32),
        pltpu.SemaphoreType.DMA((NBUF,)),
        pltpu.SemaphoreType.DMA((NBUF,)),
    ],
    compiler_params=pltpu.CompilerParams(use_tc_tiling_on_sc=False),
)(_prop_body)


BN = 1000


def _prep_body(x_ref, w1_ref, b1_ref, w2_ref, b2_ref, deg_ref,
               sa_ref, sb_ref, p9_ref, sq_ref):
    h1 = lax.dot_general(x_ref[...], w1_ref[...], (((1,), (1,)), ((), ())),
                         preferred_element_type=jnp.float32)
    h1 = jnp.maximum(h1 + b1_ref[...], 0.0)
    h = lax.dot_general(h1, w2_ref[...], (((1,), (1,)), ((), ())),
                        preferred_element_type=jnp.float32)
    h = h + b2_ref[...]
    deg = deg_ref[...]
    dinv = lax.rsqrt(deg)
    s0 = h * dinv
    sa_ref[...] = s0[:, :HC]
    sb_ref[...] = s0[:, HC:]
    p9_ref[...] = jnp.broadcast_to((1.0 - ALPHA) / deg, (BN, HC))
    sq_ref[...] = deg * dinv


def _prep(x, w1, b1, w2, b2, deg):
    return pl.pallas_call(
        _prep_body,
        grid=(N // BN,),
        in_specs=[
            pl.BlockSpec((BN, OUT), lambda i: (i, 0)),
            pl.BlockSpec((HID, OUT), lambda i: (0, 0)),
            pl.BlockSpec((1, HID), lambda i: (0, 0)),
            pl.BlockSpec((OUT, HID), lambda i: (0, 0)),
            pl.BlockSpec((1, OUT), lambda i: (0, 0)),
            pl.BlockSpec((BN, 1), lambda i: (i, 0)),
        ],
        out_specs=[
            pl.BlockSpec((BN, HC), lambda i: (i, 0)),
            pl.BlockSpec((BN, HC), lambda i: (i, 0)),
            pl.BlockSpec((BN, HC), lambda i: (i, 0)),
            pl.BlockSpec((BN, 1), lambda i: (i, 0)),
        ],
        out_shape=[
            jax.ShapeDtypeStruct((N, HC), jnp.float32),
            jax.ShapeDtypeStruct((N, HC), jnp.float32),
            jax.ShapeDtypeStruct((N, HC), jnp.float32),
            jax.ShapeDtypeStruct((N, 1), jnp.float32),
        ],
    )(x, w1, b1, w2, b2, deg)


def _final_body(fa_ref, fb_ref, sq_ref, out_ref):
    y = jnp.concatenate([fa_ref[...], fb_ref[...]], axis=1) * sq_ref[...]
    m = jnp.max(y, axis=1, keepdims=True)
    e = jnp.exp(y - m)
    lse = jnp.log(jnp.sum(e, axis=1, keepdims=True)) + m
    out_ref[...] = y - lse


def _final(fa, fb, sq):
    return pl.pallas_call(
        _final_body,
        grid=(N // BN,),
        in_specs=[
            pl.BlockSpec((BN, HC), lambda i: (i, 0)),
            pl.BlockSpec((BN, HC), lambda i: (i, 0)),
            pl.BlockSpec((BN, 1), lambda i: (i, 0)),
        ],
        out_specs=pl.BlockSpec((BN, OUT), lambda i: (i, 0)),
        out_shape=jax.ShapeDtypeStruct((N, OUT), jnp.float32),
    )(fa, fb, sq)


def kernel(x, edge_index, W1, b1, W2, b2):
    f32 = jnp.float32
    rows = edge_index[0].astype(jnp.int32).reshape(NT, EPT)
    cols = edge_index[1].astype(jnp.int32).reshape(NT, EPT)
    rows = jnp.pad(rows, ((0, 0), (0, PAD_E))).reshape(NT, NCHUNK, CHUNK)
    cols = jnp.pad(cols, ((0, 0), (0, PAD_E)),
                   constant_values=N).reshape(NT, NCHUNK, CHUNK)

    deg_full = _prop(jnp.ones((NC * NP, HC), f32), jnp.ones((NP, HC), f32),
                     jnp.zeros((NC * NP, HC), f32), rows, cols)
    deg = deg_full[:N, 0:1]

    sa, sb, p9, sq = _prep(x, W1, b1.reshape(1, HID), W2, b2.reshape(1, OUT),
                           deg)

    pad_rows = ((0, NP - N), (0, 0))
    s0 = jnp.concatenate(
        [jnp.pad(sa, pad_rows), jnp.pad(sb, pad_rows)], axis=0)
    p9b = jnp.pad(p9, pad_rows)

    s = s0
    for _ in range(K_ITERS):
        s = _prop(s, p9b, s0, rows, cols)

    return _final(s[:N], s[NP:NP + N], sq)


# speedup vs baseline: 13.0480x; 1.5077x over previous
import functools

import jax
import jax.numpy as jnp
from jax import lax
from jax.experimental import pallas as pl
from jax.experimental.pallas import tpu as pltpu
from jax.experimental.pallas import tpu_sc as plsc

N = 10000
E = 320000
HID = 256
OUT = 128
K_ITERS = 10
ALPHA = 0.1

NC = 2
NT = 16
HC = OUT // NC
EPT = E // NT
CHUNK = 128
NCHUNK = 160
PAD_E = NCHUNK * CHUNK - EPT
NP = 10240
RPT = NP // NT
NRING = 4
NGROUP = NCHUNK // NRING
NSUB = RPT // CHUNK


def _make_prop(K):
    def body(s_init, s0, p9v, rows_h, cols_h, s_out,
             G, A, rbuf, cbuf, gb0, gb1, gb2, gb3, p9t, isem, gsem, ssem):
        c = lax.axis_index("c")
        t = lax.axis_index("s")
        r0 = t * RPT
        plane = c * NP
        gbufs = (gb0, gb1, gb2, gb3)

        pltpu.sync_copy(s_init.at[pl.ds(plane + r0, RPT)],
                        G.at[pl.ds(r0, RPT)])
        pltpu.sync_copy(s_init.at[pl.ds(plane + r0, RPT)],
                        A.at[pl.ds(r0, RPT)])
        pltpu.sync_copy(p9v.at[pl.ds(r0, RPT)], p9t)
        plsc.subcore_barrier()

        def k_body(k, _):
            for b in range(NRING):
                pltpu.async_copy(rows_h.at[t, b], rbuf.at[b], isem.at[b])
                pltpu.async_copy(cols_h.at[t, b], cbuf.at[b, 0], isem.at[b])

            def group(jj, _):
                p = lax.rem(jj, 2)
                for b in range(NRING):
                    @pl.when(jj > 0)
                    def _wait_scat():
                        pltpu.make_async_copy(
                            gbufs[b], A.at[cbuf.at[b, 1 - p]],
                            ssem.at[b]).wait()
                    j = jj * NRING + b
                    pltpu.make_async_copy(
                        rows_h.at[t, j], rbuf.at[b], isem.at[b]).wait()
                    pltpu.make_async_copy(
                        cols_h.at[t, j], cbuf.at[b, p], isem.at[b]).wait()
                    pltpu.async_copy(
                        G.at[rbuf.at[b]], gbufs[b], gsem.at[b])
                for b in range(NRING):
                    j = jj * NRING + b
                    pltpu.make_async_copy(
                        G.at[rbuf.at[b]], gbufs[b], gsem.at[b]).wait()
                    pltpu.async_copy(
                        gbufs[b], A.at[cbuf.at[b, p]], ssem.at[b], add=True)

                    @pl.when(jj < NGROUP - 1)
                    def _prefetch():
                        pltpu.async_copy(
                            rows_h.at[t, j + NRING], rbuf.at[b], isem.at[b])
                        pltpu.async_copy(
                            cols_h.at[t, j + NRING], cbuf.at[b, 1 - p],
                            isem.at[b])
                return 0
            lax.fori_loop(0, NGROUP, group, 0)
            pl_last = lax.rem(NGROUP - 1, 2)
            for b in range(NRING):
                pltpu.make_async_copy(
                    gbufs[b], A.at[cbuf.at[b, pl_last]], ssem.at[b]).wait()
            plsc.subcore_barrier()

            def issue_loads(u):
                pe = u % 2
                pltpu.async_copy(A.at[pl.ds(r0 + u * CHUNK, CHUNK)],
                                 gbufs[2 * pe], gsem.at[pe])
                pltpu.async_copy(
                    s0.at[pl.ds(plane + r0 + u * CHUNK, CHUNK)],
                    gbufs[2 * pe + 1], gsem.at[pe + 2])

            def wait_loads(u):
                pe = u % 2
                pltpu.make_async_copy(A.at[pl.ds(r0 + u * CHUNK, CHUNK)],
                                      gbufs[2 * pe], gsem.at[pe]).wait()
                pltpu.make_async_copy(
                    s0.at[pl.ds(plane + r0 + u * CHUNK, CHUNK)],
                    gbufs[2 * pe + 1], gsem.at[pe + 2]).wait()

            def issue_stores(u):
                pe = u % 2
                pltpu.async_copy(gbufs[2 * pe],
                                 A.at[pl.ds(r0 + u * CHUNK, CHUNK)],
                                 ssem.at[pe])
                pltpu.async_copy(gbufs[2 * pe],
                                 G.at[pl.ds(r0 + u * CHUNK, CHUNK)],
                                 ssem.at[pe])

            def wait_stores(u):
                pe = u % 2
                pltpu.make_async_copy(gbufs[2 * pe],
                                      A.at[pl.ds(r0 + u * CHUNK, CHUNK)],
                                      ssem.at[pe]).wait()
                pltpu.make_async_copy(gbufs[2 * pe],
                                      G.at[pl.ds(r0 + u * CHUNK, CHUNK)],
                                      ssem.at[pe]).wait()

            issue_loads(0)
            for u in range(NSUB):
                pe = u % 2
                ga, gs = gbufs[2 * pe], gbufs[2 * pe + 1]
                if u + 1 < NSUB:
                    if u >= 1:
                        wait_stores(u - 1)
                    issue_loads(u + 1)
                wait_loads(u)

                def row_body(r, _, u=u, ga=ga, gs=gs):
                    pv = p9t[u * CHUNK + r, :]
                    for kk in range(HC // 16):
                        sl = pl.ds(kk * 16, 16)
                        ga[r, sl] = pv * ga[r, sl] + 0.1 * gs[r, sl]
                    return 0
                lax.fori_loop(0, CHUNK, row_body, 0)
                issue_stores(u)
            wait_stores(NSUB - 2)
            wait_stores(NSUB - 1)
            plsc.subcore_barrier()
            return 0

        lax.fori_loop(0, K, k_body, 0)
        pltpu.sync_copy(G.at[pl.ds(r0, RPT)],
                        s_out.at[pl.ds(plane + r0, RPT)])

    return pl.kernel(
        body,
        out_type=jax.ShapeDtypeStruct((NC * NP, HC), jnp.float32),
        mesh=plsc.VectorSubcoreMesh(core_axis_name="c", subcore_axis_name="s"),
        scratch_types=[
            pltpu.VMEM_SHARED((NP, HC), jnp.float32),
            pltpu.VMEM_SHARED((NP, HC), jnp.float32),
            pltpu.VMEM((NRING, CHUNK), jnp.int32),
            pltpu.VMEM((NRING, 2, CHUNK), jnp.int32),
            pltpu.VMEM((CHUNK, HC), jnp.float32),
            pltpu.VMEM((CHUNK, HC), jnp.float32),
            pltpu.VMEM((CHUNK, HC), jnp.float32),
            pltpu.VMEM((CHUNK, HC), jnp.float32),
            pltpu.VMEM((RPT, 16), jnp.float32),
            pltpu.SemaphoreType.DMA((NRING,)),
            pltpu.SemaphoreType.DMA((NRING,)),
            pltpu.SemaphoreType.DMA((NRING,)),
        ],
        compiler_params=pltpu.CompilerParams(use_tc_tiling_on_sc=False),
    )


_prop10 = _make_prop(K_ITERS)
_prop1 = _make_prop(1)


BN = 1000


def _prep_body(x_ref, w1_ref, b1_ref, w2_ref, b2_ref, deg_ref,
               sa_ref, sb_ref, p9_ref, sq_ref):
    h1 = lax.dot_general(x_ref[...], w1_ref[...], (((1,), (1,)), ((), ())),
                         preferred_element_type=jnp.float32)
    h1 = jnp.maximum(h1 + b1_ref[...], 0.0)
    h = lax.dot_general(h1, w2_ref[...], (((1,), (1,)), ((), ())),
                        preferred_element_type=jnp.float32)
    h = h + b2_ref[...]
    deg = deg_ref[...]
    dinv = lax.rsqrt(deg)
    s0 = h * dinv
    sa_ref[...] = s0[:, :HC]
    sb_ref[...] = s0[:, HC:]
    p9_ref[...] = jnp.broadcast_to((1.0 - ALPHA) / deg, (BN, 16))
    sq_ref[...] = deg * dinv


def _prep(x, w1, b1, w2, b2, deg):
    return pl.pallas_call(
        _prep_body,
        grid=(N // BN,),
        in_specs=[
            pl.BlockSpec((BN, OUT), lambda i: (i, 0)),
            pl.BlockSpec((HID, OUT), lambda i: (0, 0)),
            pl.BlockSpec((1, HID), lambda i: (0, 0)),
            pl.BlockSpec((OUT, HID), lambda i: (0, 0)),
            pl.BlockSpec((1, OUT), lambda i: (0, 0)),
            pl.BlockSpec((BN, 1), lambda i: (i, 0)),
        ],
        out_specs=[
            pl.BlockSpec((BN, HC), lambda i: (i, 0)),
            pl.BlockSpec((BN, HC), lambda i: (i, 0)),
            pl.BlockSpec((BN, 16), lambda i: (i, 0)),
            pl.BlockSpec((BN, 1), lambda i: (i, 0)),
        ],
        out_shape=[
            jax.ShapeDtypeStruct((N, HC), jnp.float32),
            jax.ShapeDtypeStruct((N, HC), jnp.float32),
            jax.ShapeDtypeStruct((N, 16), jnp.float32),
            jax.ShapeDtypeStruct((N, 1), jnp.float32),
        ],
    )(x, w1, b1, w2, b2, deg)


def _final_body(fa_ref, fb_ref, sq_ref, out_ref):
    y = jnp.concatenate([fa_ref[...], fb_ref[...]], axis=1) * sq_ref[...]
    m = jnp.max(y, axis=1, keepdims=True)
    e = jnp.exp(y - m)
    lse = jnp.log(jnp.sum(e, axis=1, keepdims=True)) + m
    out_ref[...] = y - lse


def _final(fa, fb, sq):
    return pl.pallas_call(
        _final_body,
        grid=(N // BN,),
        in_specs=[
            pl.BlockSpec((BN, HC), lambda i: (i, 0)),
            pl.BlockSpec((BN, HC), lambda i: (i, 0)),
            pl.BlockSpec((BN, 1), lambda i: (i, 0)),
        ],
        out_specs=pl.BlockSpec((BN, OUT), lambda i: (i, 0)),
        out_shape=jax.ShapeDtypeStruct((N, OUT), jnp.float32),
    )(fa, fb, sq)


def kernel(x, edge_index, W1, b1, W2, b2):
    f32 = jnp.float32
    rows = edge_index[0].astype(jnp.int32).reshape(NT, EPT)
    cols = edge_index[1].astype(jnp.int32).reshape(NT, EPT)
    rows = jnp.pad(rows, ((0, 0), (0, PAD_E))).reshape(NT, NCHUNK, CHUNK)
    cols = jnp.pad(cols, ((0, 0), (0, PAD_E)),
                   constant_values=N).reshape(NT, NCHUNK, CHUNK)

    deg_full = _prop1(jnp.ones((NC * NP, HC), f32),
                      jnp.zeros((NC * NP, HC), f32),
                      jnp.ones((NP, 16), f32), rows, cols)
    deg = deg_full[:N, 0:1]

    sa, sb, p9, sq = _prep(x, W1, b1.reshape(1, HID), W2, b2.reshape(1, OUT),
                           deg)

    pad_rows = ((0, NP - N), (0, 0))
    s0 = jnp.concatenate(
        [jnp.pad(sa, pad_rows), jnp.pad(sb, pad_rows)], axis=0)
    p9v = jnp.pad(p9, pad_rows)

    s = _prop10(s0, s0, p9v, rows, cols)

    return _final(s[:N], s[NP:NP + N], sq)
